# Initial kernel scaffold; baseline (speedup 1.0000x reference)
import jax
import jax.numpy as jnp
from jax.experimental import pallas as pl


def kernel(species_tokens, items_tokens, abilities_tokens, actions_tokens, species_table, items_table, abilities_table, actions_table, W_combine):
    raise NotImplementedError("write your pallas kernel here")


# speedup vs baseline: 1.1174x; 1.1174x over previous
import functools

import jax
import jax.numpy as jnp
from jax import lax
from jax.experimental import pallas as pl
from jax.experimental.pallas import tpu as pltpu
from jax.experimental.pallas import tpu_sc as plsc

E = 64
B = 16384
L = 12
TOK = B * L
NC, NS = 2, 16
NW = NC * NS
PER_W = TOK // NW
C = 128
N_CHUNKS = PER_W // C

BLK = 2048
N_BLKS = TOK // BLK


def _sc_gather_sum(ts, ti, ta, tact, tab_s, tab_i, tab_a, tab_act):
    mesh = plsc.VectorSubcoreMesh(core_axis_name="c", subcore_axis_name="s")

    @functools.partial(
        pl.kernel,
        mesh=mesh,
        out_type=jax.ShapeDtypeStruct((TOK, E), jnp.float32),
        compiler_params=pltpu.CompilerParams(use_tc_tiling_on_sc=False),
        scratch_types=[
            pltpu.VMEM((C,), jnp.int32),
            pltpu.VMEM((C,), jnp.int32),
            pltpu.VMEM((C,), jnp.int32),
            pltpu.VMEM((C,), jnp.int32),
            pltpu.VMEM((C, E), jnp.float32),
            pltpu.VMEM((C, E), jnp.float32),
            pltpu.VMEM((C, E), jnp.float32),
            pltpu.VMEM((C, E), jnp.float32),
            pltpu.SemaphoreType.DMA,
        ],
    )
    def k(ts_h, ti_h, ta_h, tact_h, tabs_h, tabi_h, taba_h, tabact_h, out_h,
          i0, i1, i2, i3, b0, b1, b2, b3, sem):
        wid = lax.axis_index("s") * NC + lax.axis_index("c")
        base_w = wid * PER_W

        def chunk(g, carry):
            base = base_w + g * C
            pltpu.sync_copy(ts_h.at[pl.ds(base, C)], i0)
            pltpu.sync_copy(ti_h.at[pl.ds(base, C)], i1)
            pltpu.sync_copy(ta_h.at[pl.ds(base, C)], i2)
            pltpu.sync_copy(tact_h.at[pl.ds(base, C)], i3)
            c0 = pltpu.async_copy(tabs_h.at[i0], b0, sem)
            c1 = pltpu.async_copy(tabi_h.at[i1], b1, sem)
            c2 = pltpu.async_copy(taba_h.at[i2], b2, sem)
            c3 = pltpu.async_copy(tabact_h.at[i3], b3, sem)
            c0.wait()
            c1.wait()
            c2.wait()
            c3.wait()

            def row(i, carry2):
                for j in range(E // 16):
                    sl = pl.ds(j * 16, 16)
                    b0[i, sl] = b0[i, sl] + b1[i, sl] + b2[i, sl] + b3[i, sl]
                return carry2

            lax.fori_loop(0, C, row, 0)
            pltpu.sync_copy(b0, out_h.at[pl.ds(base, C)])
            return carry

        lax.fori_loop(0, N_CHUNKS, chunk, 0)

    return k(ts, ti, ta, tact, tab_s, tab_i, tab_a, tab_act)


def _tc_body(x_ref, ts_ref, ti_ref, ta_ref, tact_ref, r0_ref, w_ref, o_ref):
    wv = w_ref[...]
    r0w = jnp.dot(r0_ref[...], wv)
    xw = jnp.dot(x_ref[...], wv)
    ms = (ts_ref[...][0] == 0).astype(jnp.float32)
    mi = (ti_ref[...][0] == 0).astype(jnp.float32)
    ma = (ta_ref[...][0] == 0).astype(jnp.float32)
    mact = (tact_ref[...][0] == 0).astype(jnp.float32)
    z = jnp.concatenate(
        [ms, mi, ma, mact, jnp.zeros((4, BLK), jnp.float32)], axis=0)
    corr = lax.dot_general(z, r0w, (((0,), (0,)), ((), ())))
    o_ref[...] = jnp.maximum(xw - corr, 0.0)


def _tc_project(x, ts3, ti3, ta3, tact3, r0pad, w):
    tok_spec = pl.BlockSpec((1, 1, BLK), lambda i: (i, 0, 0))
    return pl.pallas_call(
        _tc_body,
        grid=(N_BLKS,),
        in_specs=[
            pl.BlockSpec((BLK, E), lambda i: (i, 0)),
            tok_spec, tok_spec, tok_spec, tok_spec,
            pl.BlockSpec((8, E), lambda i: (0, 0)),
            pl.BlockSpec((E, E), lambda i: (0, 0)),
        ],
        out_specs=pl.BlockSpec((BLK, E), lambda i: (i, 0)),
        out_shape=jax.ShapeDtypeStruct((TOK, E), jnp.float32),
    )(x, ts3, ti3, ta3, tact3, r0pad, w)


def kernel(species_tokens, items_tokens, abilities_tokens, actions_tokens,
           species_table, items_table, abilities_table, actions_table,
           W_combine):
    ts = species_tokens.reshape(-1).astype(jnp.int32)
    ti = items_tokens.reshape(-1).astype(jnp.int32)
    ta = abilities_tokens.reshape(-1).astype(jnp.int32)
    tact = actions_tokens.reshape(-1).astype(jnp.int32)

    combined = _sc_gather_sum(ts, ti, ta, tact, species_table, items_table,
                              abilities_table, actions_table)

    r0 = jnp.concatenate([
        species_table[0:1], items_table[0:1],
        abilities_table[0:1], actions_table[0:1],
        jnp.zeros((4, E), jnp.float32),
    ], axis=0)

    ts3 = ts.reshape(N_BLKS, 1, BLK)
    ti3 = ti.reshape(N_BLKS, 1, BLK)
    ta3 = ta.reshape(N_BLKS, 1, BLK)
    tact3 = tact.reshape(N_BLKS, 1, BLK)

    out = _tc_project(combined, ts3, ti3, ta3, tact3, r0, W_combine)
    return out.reshape(B, L, E)


# speedup vs baseline: 1.1643x; 1.0420x over previous
import functools

import jax
import jax.numpy as jnp
from jax import lax
from jax.experimental import pallas as pl
from jax.experimental.pallas import tpu as pltpu
from jax.experimental.pallas import tpu_sc as plsc

E = 64
B = 16384
L = 12
TOK = B * L
NC, NS = 2, 16
NW = NC * NS
PER_W = TOK // NW
C = 128
CP = C // 2
N_CHUNKS = PER_W // C

BLK = 1024
N_BLKS = (TOK // 2) // BLK


def _sc_gather_sum(ts2, ti2, ta2, tact2, tab_s, tab_i, tab_a, tab_act):
    mesh = plsc.VectorSubcoreMesh(core_axis_name="c", subcore_axis_name="s")

    @functools.partial(
        pl.kernel,
        mesh=mesh,
        out_type=jax.ShapeDtypeStruct((TOK // 2, 2 * E), jnp.float32),
        compiler_params=pltpu.CompilerParams(use_tc_tiling_on_sc=False),
        scratch_types=[
            pltpu.VMEM((N_CHUNKS, C), jnp.int32),
            pltpu.VMEM((N_CHUNKS, C), jnp.int32),
            pltpu.VMEM((N_CHUNKS, C), jnp.int32),
            pltpu.VMEM((N_CHUNKS, C), jnp.int32),
            pltpu.VMEM((C, E), jnp.float32),
            pltpu.VMEM((C, E), jnp.float32),
            pltpu.VMEM((C, E), jnp.float32),
            pltpu.VMEM((C, E), jnp.float32),
            pltpu.VMEM((C, E), jnp.float32),
            pltpu.VMEM((C, E), jnp.float32),
            pltpu.VMEM((C, E), jnp.float32),
            pltpu.VMEM((C, E), jnp.float32),
            pltpu.VMEM((CP, 2 * E), jnp.float32),
            pltpu.VMEM((CP, 2 * E), jnp.float32),
            pltpu.SemaphoreType.DMA,
            pltpu.SemaphoreType.DMA,
            pltpu.SemaphoreType.DMA,
            pltpu.SemaphoreType.DMA,
        ],
    )
    def k(ts_h, ti_h, ta_h, tact_h, tabs_h, tabi_h, taba_h, tabact_h, out_h,
          s0, s1, s2, s3,
          b00, b01, b02, b03, b10, b11, b12, b13,
          o0, o1,
          sg0, sg1, so0, so1):
        wid = lax.axis_index("s") * NC + lax.axis_index("c")
        base_chunk = wid * N_CHUNKS
        out_base = wid * (PER_W // 2)

        slabs = (s0, s1, s2, s3)
        tabs = (tabs_h, tabi_h, taba_h, tabact_h)
        bufs = ((b00, b01, b02, b03), (b10, b11, b12, b13))
        obufs = (o0, o1)
        gsems = (sg0, sg1)
        osems = (so0, so1)

        pltpu.sync_copy(ts_h.at[pl.ds(base_chunk, N_CHUNKS)], s0)
        pltpu.sync_copy(ti_h.at[pl.ds(base_chunk, N_CHUNKS)], s1)
        pltpu.sync_copy(ta_h.at[pl.ds(base_chunk, N_CHUNKS)], s2)
        pltpu.sync_copy(tact_h.at[pl.ds(base_chunk, N_CHUNKS)], s3)

        def fire(g, slot):
            for t in range(4):
                pltpu.async_copy(tabs[t].at[slabs[t].at[g]],
                                 bufs[slot][t], gsems[slot])

        def wait_gathers(slot):
            for t in range(4):
                pltpu.make_async_copy(tabs[t].at[pl.ds(0, C)],
                                      bufs[slot][t], gsems[slot]).wait()

        def wait_out(slot):
            pltpu.make_async_copy(obufs[slot],
                                  out_h.at[pl.ds(0, CP)], osems[slot]).wait()

        def do_sum(slot):
            bt = bufs[slot]
            ob = obufs[slot]

            def row(p, carry):
                for j in range(8):
                    r = 2 * p + (j // 4)
                    sl = pl.ds((j % 4) * 16, 16)
                    ob[p, pl.ds(j * 16, 16)] = (
                        bt[0][r, sl] + bt[1][r, sl] + bt[2][r, sl]
                        + bt[3][r, sl])
                return carry

            lax.fori_loop(0, CP, row, 0)

        fire(0, 0)

        def chunk_pair(gg, carry):
            for slot in range(2):
                g = 2 * gg + slot

                @pl.when(g + 1 < N_CHUNKS)
                def _():
                    fire(g + 1, 1 - slot)

                wait_gathers(slot)

                @pl.when(g >= 2)
                def _():
                    wait_out(slot)

                do_sum(slot)
                pltpu.async_copy(
                    obufs[slot], out_h.at[pl.ds(out_base + g * CP, CP)],
                    osems[slot])
            return carry

        lax.fori_loop(0, N_CHUNKS // 2, chunk_pair, 0)
        wait_out(0)
        wait_out(1)

    return k(ts2, ti2, ta2, tact2, tab_s, tab_i, tab_a, tab_act)


def _tc_body(x_ref, tse_ref, tso_ref, tie_ref, tio_ref, tae_ref, tao_ref,
             tacte_ref, tacto_ref, r0_ref, w_ref, o_ref):
    wv = w_ref[...]
    zz = jnp.zeros((E, E), jnp.float32)
    w2 = jnp.concatenate([
        jnp.concatenate([wv, zz], axis=1),
        jnp.concatenate([zz, wv], axis=1),
    ], axis=0)
    xw = jnp.dot(x_ref[...], w2)

    r0w = jnp.dot(r0_ref[...], wv)
    z8 = jnp.zeros((8, E), jnp.float32)
    r_left = jnp.concatenate([r0w, z8], axis=1)
    r_right = jnp.concatenate([z8, r0w], axis=1)

    def masks(refs):
        ms = [(r[...][0] == 0).astype(jnp.float32) for r in refs]
        return jnp.concatenate(ms + [jnp.zeros((4, BLK), jnp.float32)], axis=0)

    ze = masks([tse_ref, tie_ref, tae_ref, tacte_ref])
    zo = masks([tso_ref, tio_ref, tao_ref, tacto_ref])
    corr = (lax.dot_general(ze, r_left, (((0,), (0,)), ((), ())))
            + lax.dot_general(zo, r_right, (((0,), (0,)), ((), ()))))
    o_ref[...] = jnp.maximum(xw - corr, 0.0)


def _tc_project(x, toks8, r0pad, w):
    tok_spec = pl.BlockSpec((1, 1, BLK), lambda i: (i, 0, 0))
    return pl.pallas_call(
        _tc_body,
        grid=(N_BLKS,),
        in_specs=[
            pl.BlockSpec((BLK, 2 * E), lambda i: (i, 0)),
            tok_spec, tok_spec, tok_spec, tok_spec,
            tok_spec, tok_spec, tok_spec, tok_spec,
            pl.BlockSpec((8, E), lambda i: (0, 0)),
            pl.BlockSpec((E, E), lambda i: (0, 0)),
        ],
        out_specs=pl.BlockSpec((BLK, 2 * E), lambda i: (i, 0)),
        out_shape=jax.ShapeDtypeStruct((TOK // 2, 2 * E), jnp.float32),
    )(x, *toks8, r0pad, w)


def kernel(species_tokens, items_tokens, abilities_tokens, actions_tokens,
           species_table, items_table, abilities_table, actions_table,
           W_combine):
    toks = [t.reshape(-1).astype(jnp.int32) for t in
            (species_tokens, items_tokens, abilities_tokens, actions_tokens)]
    toks2d = [t.reshape(NW * N_CHUNKS, C) for t in toks]

    combined = _sc_gather_sum(*toks2d, species_table, items_table,
                              abilities_table, actions_table)

    r0 = jnp.concatenate([
        species_table[0:1], items_table[0:1],
        abilities_table[0:1], actions_table[0:1],
        jnp.zeros((4, E), jnp.float32),
    ], axis=0)

    toks8 = []
    for t in toks:
        tp = t.reshape(-1, 2)
        toks8.append(tp[:, 0].reshape(N_BLKS, 1, BLK))
        toks8.append(tp[:, 1].reshape(N_BLKS, 1, BLK))

    out = _tc_project(combined, toks8, r0, W_combine)
    return out.reshape(B, L, E)


# speedup vs baseline: 1.7649x; 1.5159x over previous
import functools

import jax
import jax.numpy as jnp
from jax import lax
from jax.experimental import pallas as pl
from jax.experimental.pallas import tpu as pltpu
from jax.experimental.pallas import tpu_sc as plsc

E = 64
B = 16384
L = 12
TOK = B * L
NC, NS = 2, 16
NW = NC * NS
PER_W = TOK // NW
C = 128
CP = C // 2
N_CHUNKS = PER_W // C

BLK = 2048
N_BLKS = (TOK // 2) // BLK


def _sc_gather_sum(ts2, ti2, ta2, tact2, tab_s, tab_i, tab_a, tab_act):
    mesh = plsc.VectorSubcoreMesh(core_axis_name="c", subcore_axis_name="s")

    @functools.partial(
        pl.kernel,
        mesh=mesh,
        out_type=jax.ShapeDtypeStruct((TOK // 2, 2 * E), jnp.float32),
        compiler_params=pltpu.CompilerParams(use_tc_tiling_on_sc=False),
        scratch_types=[
            pltpu.VMEM((N_CHUNKS, C), jnp.int32),
            pltpu.VMEM((N_CHUNKS, C), jnp.int32),
            pltpu.VMEM((N_CHUNKS, C), jnp.int32),
            pltpu.VMEM((N_CHUNKS, C), jnp.int32),
            pltpu.VMEM((C, E), jnp.float32),
            pltpu.VMEM((C, E), jnp.float32),
            pltpu.VMEM((C, E), jnp.float32),
            pltpu.VMEM((C, E), jnp.float32),
            pltpu.VMEM((C, E), jnp.float32),
            pltpu.VMEM((C, E), jnp.float32),
            pltpu.VMEM((C, E), jnp.float32),
            pltpu.VMEM((C, E), jnp.float32),
            pltpu.VMEM((CP, 2 * E), jnp.float32),
            pltpu.VMEM((CP, 2 * E), jnp.float32),
            pltpu.SemaphoreType.DMA,
            pltpu.SemaphoreType.DMA,
            pltpu.SemaphoreType.DMA,
            pltpu.SemaphoreType.DMA,
        ],
    )
    def k(ts_h, ti_h, ta_h, tact_h, tabs_h, tabi_h, taba_h, tabact_h, out_h,
          s0, s1, s2, s3,
          b00, b01, b02, b03, b10, b11, b12, b13,
          o0, o1,
          sg0, sg1, so0, so1):
        wid = lax.axis_index("s") * NC + lax.axis_index("c")
        base_chunk = wid * N_CHUNKS
        out_base = wid * (PER_W // 2)

        slabs = (s0, s1, s2, s3)
        tabs = (tabs_h, tabi_h, taba_h, tabact_h)
        bufs = ((b00, b01, b02, b03), (b10, b11, b12, b13))
        obufs = (o0, o1)
        gsems = (sg0, sg1)
        osems = (so0, so1)

        pltpu.sync_copy(ts_h.at[pl.ds(base_chunk, N_CHUNKS)], s0)
        pltpu.sync_copy(ti_h.at[pl.ds(base_chunk, N_CHUNKS)], s1)
        pltpu.sync_copy(ta_h.at[pl.ds(base_chunk, N_CHUNKS)], s2)
        pltpu.sync_copy(tact_h.at[pl.ds(base_chunk, N_CHUNKS)], s3)

        def fire(g, slot):
            for t in range(4):
                pltpu.async_copy(tabs[t].at[slabs[t].at[g]],
                                 bufs[slot][t], gsems[slot])

        def wait_gathers(slot):
            for t in range(4):
                pltpu.make_async_copy(tabs[t].at[pl.ds(0, C)],
                                      bufs[slot][t], gsems[slot]).wait()

        def wait_out(slot):
            pltpu.make_async_copy(obufs[slot],
                                  out_h.at[pl.ds(0, CP)], osems[slot]).wait()

        def do_sum(g, slot):
            bt = bufs[slot]
            ob = obufs[slot]

            def group(k, carry):
                mv = [jnp.where(slabs[t][g, pl.ds(k * 16, 16)] == 0, 0.0, 1.0)
                      for t in range(4)]
                for j in range(16):
                    r = 16 * k + j
                    p = 8 * k + (j // 2)
                    off = (j % 2) * E
                    for q in range(4):
                        sl = pl.ds(q * 16, 16)
                        ob[p, pl.ds(off + q * 16, 16)] = (
                            mv[0][j] * bt[0][r, sl] + mv[1][j] * bt[1][r, sl]
                            + mv[2][j] * bt[2][r, sl]
                            + mv[3][j] * bt[3][r, sl])
                return carry

            lax.fori_loop(0, C // 16, group, 0)

        fire(0, 0)

        def chunk_pair(gg, carry):
            for slot in range(2):
                g = 2 * gg + slot

                @pl.when(g + 1 < N_CHUNKS)
                def _():
                    fire(g + 1, 1 - slot)

                wait_gathers(slot)

                @pl.when(g >= 2)
                def _():
                    wait_out(slot)

                do_sum(g, slot)
                pltpu.async_copy(
                    obufs[slot], out_h.at[pl.ds(out_base + g * CP, CP)],
                    osems[slot])
            return carry

        lax.fori_loop(0, N_CHUNKS // 2, chunk_pair, 0)
        wait_out(0)
        wait_out(1)

    return k(ts2, ti2, ta2, tact2, tab_s, tab_i, tab_a, tab_act)


def _tc_body(x_ref, w_ref, o_ref):
    wv = w_ref[...]
    zz = jnp.zeros((E, E), jnp.float32)
    w2 = jnp.concatenate([
        jnp.concatenate([wv, zz], axis=1),
        jnp.concatenate([zz, wv], axis=1),
    ], axis=0)
    o_ref[...] = jnp.maximum(jnp.dot(x_ref[...], w2), 0.0)


def _tc_project(x, w):
    return pl.pallas_call(
        _tc_body,
        grid=(N_BLKS,),
        in_specs=[
            pl.BlockSpec((BLK, 2 * E), lambda i: (i, 0)),
            pl.BlockSpec((E, E), lambda i: (0, 0)),
        ],
        out_specs=pl.BlockSpec((BLK, 2 * E), lambda i: (i, 0)),
        out_shape=jax.ShapeDtypeStruct((TOK // 2, 2 * E), jnp.float32),
    )(x, w)


def kernel(species_tokens, items_tokens, abilities_tokens, actions_tokens,
           species_table, items_table, abilities_table, actions_table,
           W_combine):
    toks = [t.reshape(-1).astype(jnp.int32) for t in
            (species_tokens, items_tokens, abilities_tokens, actions_tokens)]
    toks2d = [t.reshape(NW * N_CHUNKS, C) for t in toks]

    combined = _sc_gather_sum(*toks2d, species_table, items_table,
                              abilities_table, actions_table)

    out = _tc_project(combined, W_combine)
    return out.reshape(B, L, E)
